# baseline (speedup 1.0000x reference)
import functools

import jax
import jax.numpy as jnp
from jax import lax
from jax.experimental import pallas as pl
from jax.experimental.pallas import tpu as pltpu
from jax.experimental.pallas import tpu_sc as plsc

NUSER = 1000000
DIM = 64
B_TOTAL = 4096
L_TOTAL = 200
LANES = 16
NUM_CORES = 2
NUM_SUBCORES = 16
D_PER_CORE = DIM // NUM_CORES
CHUNK = 2048
GATHER_IDX = 128


def _sc_kernel(ids_hbm, tg_hbm, emb_hbm, tab_hbm, out_hbm,
               ids_v, tg_v, emb_v, shift_v, row_sp, sem_g, sem_e):
    c = lax.axis_index("c")
    s = lax.axis_index("s")
    n_total = L_TOTAL * B_TOTAL
    n_per_tile = n_total // NUM_SUBCORES
    n0 = s * n_per_tile
    nchunks = n_per_tile // CHUNK

    pltpu.sync_copy(ids_hbm.at[pl.ds(n0, n_per_tile)], ids_v)

    def d_body(dd, carry):
        d = c * D_PER_CORE + dd
        k_hi = d // 8
        d_lo = d - k_hi * 8

        @pl.when(s == 0)
        def _():
            pltpu.sync_copy(tab_hbm.at[d], row_sp)

        plsc.subcore_barrier()

        def chunk_body(ch, carry2):
            gn = n0 + ch * CHUNK
            l = gn // B_TOTAL
            b_hi0 = (gn - l * B_TOTAL) // 128

            emb_cp = pltpu.async_copy(
                emb_hbm.at[l, k_hi, pl.ds(b_hi0, LANES),
                           pl.ds(d_lo * 128, 128)],
                emb_v, sem_e)
            tg_cp = pltpu.async_copy(
                tg_hbm.at[pl.ds(gn, CHUNK)], tg_v, sem_e)
            for g in range(CHUNK // GATHER_IDX):
                pltpu.async_copy(
                    row_sp.at[ids_v.at[pl.ds(ch * CHUNK + g * GATHER_IDX,
                                             GATHER_IDX)]],
                    shift_v.at[pl.ds(g * GATHER_IDX, GATHER_IDX)],
                    sem_g)
            emb_cp.wait()
            tg_cp.wait()
            for g in range(CHUNK // GATHER_IDX):
                pltpu.make_async_copy(
                    row_sp.at[ids_v.at[pl.ds(ch * CHUNK + g * GATHER_IDX,
                                             GATHER_IDX)]],
                    shift_v.at[pl.ds(g * GATHER_IDX, GATHER_IDX)],
                    sem_g).wait()

            def vec_body(i, carry3):
                r = i // 8
                col = (i - r * 8) * LANES
                e = emb_v[r, pl.ds(col, LANES)]
                sh = shift_v[pl.ds(i * LANES, LANES)]
                t = tg_v[pl.ds(i * LANES, LANES)]
                sig = 1.0 / (1.0 + jnp.exp(-(t * sh)))
                emb_v[r, pl.ds(col, LANES)] = e * (1.0 + sig)
                return carry3

            lax.fori_loop(0, CHUNK // LANES, vec_body, 0, unroll=8)
            pltpu.sync_copy(
                emb_v,
                out_hbm.at[l, k_hi, pl.ds(b_hi0, LANES),
                           pl.ds(d_lo * 128, 128)])
            return carry2

        lax.fori_loop(0, nchunks, chunk_body, 0)
        plsc.subcore_barrier()
        return carry

    lax.fori_loop(0, D_PER_CORE, d_body, 0)


@jax.jit
def _dspp_sc(ids_t, tg_t, emb_4d, tab_t):
    n_total = ids_t.shape[0]
    mesh = plsc.VectorSubcoreMesh(core_axis_name="c", subcore_axis_name="s")
    n_per_tile = n_total // NUM_SUBCORES
    run = pl.kernel(
        _sc_kernel,
        out_type=jax.ShapeDtypeStruct(
            (L_TOTAL, DIM // 8, B_TOTAL // 128, 1024), jnp.float32),
        mesh=mesh,
        scratch_types=[
            pltpu.VMEM((n_per_tile,), jnp.int32),
            pltpu.VMEM((CHUNK,), jnp.float32),
            pltpu.VMEM((LANES, 128), jnp.float32),
            pltpu.VMEM((CHUNK,), jnp.float32),
            pltpu.VMEM_SHARED((NUSER,), jnp.float32),
            pltpu.SemaphoreType.DMA,
            pltpu.SemaphoreType.DMA,
        ],
        compiler_params=pltpu.CompilerParams(use_tc_tiling_on_sc=False),
    )
    return run(ids_t, tg_t, emb_4d, tab_t)


def kernel(id, emb, time_gap, user_shift_table):
    B, L = id.shape
    n = B * L
    emb_4d = (emb.transpose(1, 2, 0)
              .reshape(L, DIM // 8, 8, B // 128, 128)
              .transpose(0, 1, 3, 2, 4)
              .reshape(L, DIM // 8, B // 128, 1024))
    out_4d = _dspp_sc(
        id.T.reshape(n).astype(jnp.int32),
        time_gap.T.reshape(n),
        emb_4d,
        user_shift_table.T,
    )
    out = (out_4d.reshape(L, DIM // 8, B // 128, 8, 128)
           .transpose(0, 1, 3, 2, 4)
           .reshape(L, DIM, B)
           .transpose(2, 0, 1))
    return out


# speedup vs baseline: 2.1024x; 2.1024x over previous
import jax
import jax.numpy as jnp
from jax import lax
from jax.experimental import pallas as pl
from jax.experimental.pallas import tpu as pltpu
from jax.experimental.pallas import tpu_sc as plsc

NUSER = 1000000
DIM = 64
B_TOTAL = 4096
L_TOTAL = 200
LANES = 16
NUM_CORES = 2
NUM_SUBCORES = 16
NW = NUM_CORES * NUM_SUBCORES
CHUNK = 512


def _sc_kernel(ids_hbm, tg_hbm, emb_hbm, tab_hbm, out_hbm,
               idx_v, tg_v, rows_v, emb_v, sem_g, sem_e):
    c = lax.axis_index("c")
    s = lax.axis_index("s")
    wid = s * NUM_CORES + c
    n_total = L_TOTAL * B_TOTAL
    n_per_w = n_total // NW
    nchunks = n_per_w // CHUNK
    n0 = wid * n_per_w

    iota = lax.iota(jnp.int32, LANES)

    def chunk_body(ch, carry):
        gn = n0 + ch * CHUNK
        l = gn // B_TOTAL
        b_hi0 = (gn - l * B_TOTAL) // 128

        pltpu.sync_copy(ids_hbm.at[pl.ds(gn, CHUNK)], idx_v)
        gather = pltpu.async_copy(tab_hbm.at[idx_v], rows_v, sem_g)
        emb_cp = pltpu.async_copy(
            emb_hbm.at[l, pl.ds(0, DIM // 8), pl.ds(b_hi0, CHUNK // 128)],
            emb_v, sem_e)
        tg_cp = pltpu.async_copy(tg_hbm.at[pl.ds(gn, CHUNK)], tg_v, sem_e)
        gather.wait()
        emb_cp.wait()
        tg_cp.wait()

        def pos_body(j, carry2):
            p = j * LANES
            t = tg_v[pl.ds(p, LANES)]
            row_idx = p + iota
            blk = j // 8
            off = (j - blk * 8) * LANES
            for k_hi in range(DIM // 8):
                for d_lo in range(8):
                    sh = plsc.load_gather(
                        rows_v, [row_idx, jnp.full((LANES,), k_hi * 8 + d_lo,
                                                   jnp.int32)])
                    e = emb_v[k_hi, blk, pl.ds(d_lo * 128 + off, LANES)]
                    sig = 1.0 / (1.0 + jnp.exp(-(t * sh)))
                    emb_v[k_hi, blk, pl.ds(d_lo * 128 + off, LANES)] = (
                        e * (1.0 + sig))
            return carry2

        lax.fori_loop(0, CHUNK // LANES, pos_body, 0)
        pltpu.sync_copy(
            emb_v,
            out_hbm.at[l, pl.ds(0, DIM // 8), pl.ds(b_hi0, CHUNK // 128)])
        return carry

    lax.fori_loop(0, nchunks, chunk_body, 0)


@jax.jit
def _dspp_sc(ids_t, tg_t, emb_4d, table):
    mesh = plsc.VectorSubcoreMesh(core_axis_name="c", subcore_axis_name="s")
    run = pl.kernel(
        _sc_kernel,
        out_type=jax.ShapeDtypeStruct(
            (L_TOTAL, DIM // 8, B_TOTAL // 128, 1024), jnp.float32),
        mesh=mesh,
        scratch_types=[
            pltpu.VMEM((CHUNK,), jnp.int32),
            pltpu.VMEM((CHUNK,), jnp.float32),
            pltpu.VMEM((CHUNK, DIM), jnp.float32),
            pltpu.VMEM((DIM // 8, CHUNK // 128, 1024), jnp.float32),
            pltpu.SemaphoreType.DMA,
            pltpu.SemaphoreType.DMA,
        ],
        compiler_params=pltpu.CompilerParams(
            use_tc_tiling_on_sc=False, needs_layout_passes=False),
    )
    return run(ids_t, tg_t, emb_4d, table)


def kernel(id, emb, time_gap, user_shift_table):
    B, L = id.shape
    n = B * L
    emb_4d = (emb.transpose(1, 2, 0)
              .reshape(L, DIM // 8, 8, B // 128, 128)
              .transpose(0, 1, 3, 2, 4)
              .reshape(L, DIM // 8, B // 128, 1024))
    out_4d = _dspp_sc(
        id.T.reshape(n).astype(jnp.int32),
        time_gap.T.reshape(n),
        emb_4d,
        user_shift_table,
    )
    out = (out_4d.reshape(L, DIM // 8, B // 128, 8, 128)
           .transpose(0, 1, 3, 2, 4)
           .reshape(L, DIM, B)
           .transpose(2, 0, 1))
    return out


# speedup vs baseline: 4.7165x; 2.2433x over previous
import functools

import jax
import jax.numpy as jnp
from jax import lax
from jax.experimental import pallas as pl
from jax.experimental.pallas import tpu as pltpu
from jax.experimental.pallas import tpu_sc as plsc

DIM = 64
LANES = 16
NUM_CORES = 2
NUM_SUBCORES = 16
NW = NUM_CORES * NUM_SUBCORES
CHUNK = 400


def _sc_kernel(n_rows, ids_hbm, tg_hbm, emb_hbm, table_hbm, out_hbm,
               idx0, idx1, tgv0, tgv1, sh0, sh1, em0, em1,
               sg0, sg1, sl0, sl1, so0, so1):
    idx = (idx0, idx1)
    tgv = (tgv0, tgv1)
    sh = (sh0, sh1)
    em = (em0, em1)
    sg = (sg0, sg1)
    sl = (sl0, sl1)
    so = (so0, so1)
    wid = lax.axis_index("s") * NUM_CORES + lax.axis_index("c")
    rows_per_w = n_rows // NW
    nchunks = rows_per_w // CHUNK
    wbase = wid * rows_per_w

    def issue(ci, b):
        base = wbase + ci * CHUNK
        pltpu.sync_copy(ids_hbm.at[pl.ds(base, CHUNK)], idx[b])
        pltpu.async_copy(table_hbm.at[idx[b]], sh[b], sg[b])
        pltpu.async_copy(tg_hbm.at[pl.ds(base, CHUNK)], tgv[b], sl[b])
        pltpu.async_copy(emb_hbm.at[pl.ds(base, CHUNK)], em[b], sl[b])

    def wait_in(ci, b):
        base = wbase + ci * CHUNK
        pltpu.make_async_copy(table_hbm.at[idx[b]], sh[b], sg[b]).wait()
        pltpu.make_async_copy(tg_hbm.at[pl.ds(base, CHUNK)], tgv[b], sl[b]).wait()
        pltpu.make_async_copy(emb_hbm.at[pl.ds(base, CHUNK)], em[b], sl[b]).wait()

    def wait_out(ci, b):
        base = wbase + ci * CHUNK
        pltpu.make_async_copy(em[b], out_hbm.at[pl.ds(base, CHUNK)], so[b]).wait()

    def compute(b):
        def block_body(rb, carry2):
            r0 = rb * LANES
            tgl = tgv[b][pl.ds(r0, LANES)]
            for j in range(LANES):
                i = r0 + j
                tgb = jnp.full((LANES,), tgl[j], jnp.float32)
                for k in range(DIM // LANES):
                    s = sh[b][i, pl.ds(k * LANES, LANES)]
                    e = em[b][i, pl.ds(k * LANES, LANES)]
                    sig = 1.0 / (1.0 + jnp.exp(-(tgb * s)))
                    em[b][i, pl.ds(k * LANES, LANES)] = e * (1.0 + sig)
            return carry2

        lax.fori_loop(0, CHUNK // LANES, block_body, 0)

    issue(0, 0)

    def pair_body(i2, carry):
        for b in range(2):
            ci = i2 * 2 + b
            nb = 1 - b

            @pl.when(ci >= 1)
            def _():
                wait_out(ci - 1, nb)

            @pl.when(ci + 1 < nchunks)
            def _():
                issue(ci + 1, nb)

            wait_in(ci, b)
            compute(b)
            base = wbase + ci * CHUNK
            pltpu.async_copy(em[b], out_hbm.at[pl.ds(base, CHUNK)], so[b])
        return carry

    lax.fori_loop(0, nchunks // 2, pair_body, 0)
    wait_out(nchunks - 1, (nchunks - 1) % 2)


@jax.jit
def _dspp_sc(ids_flat, tg_flat, emb_flat, table):
    n_rows = ids_flat.shape[0]
    mesh = plsc.VectorSubcoreMesh(core_axis_name="c", subcore_axis_name="s")
    run = pl.kernel(
        functools.partial(_sc_kernel, n_rows),
        out_type=jax.ShapeDtypeStruct((n_rows, DIM), jnp.float32),
        mesh=mesh,
        scratch_types=[
            pltpu.VMEM((CHUNK,), jnp.int32),
            pltpu.VMEM((CHUNK,), jnp.int32),
            pltpu.VMEM((CHUNK,), jnp.float32),
            pltpu.VMEM((CHUNK,), jnp.float32),
            pltpu.VMEM((CHUNK, DIM), jnp.float32),
            pltpu.VMEM((CHUNK, DIM), jnp.float32),
            pltpu.VMEM((CHUNK, DIM), jnp.float32),
            pltpu.VMEM((CHUNK, DIM), jnp.float32),
            pltpu.SemaphoreType.DMA,
            pltpu.SemaphoreType.DMA,
            pltpu.SemaphoreType.DMA,
            pltpu.SemaphoreType.DMA,
            pltpu.SemaphoreType.DMA,
            pltpu.SemaphoreType.DMA,
        ],
        compiler_params=pltpu.CompilerParams(use_tc_tiling_on_sc=False),
    )
    return run(ids_flat, tg_flat, emb_flat, table)


def kernel(id, emb, time_gap, user_shift_table):
    B, L = id.shape
    n = B * L
    out = _dspp_sc(
        id.reshape(n).astype(jnp.int32),
        time_gap.reshape(n),
        emb.reshape(n, DIM),
        user_shift_table,
    )
    return out.reshape(B, L, DIM)


# speedup vs baseline: 4.9332x; 1.0460x over previous
import jax
import jax.numpy as jnp
from jax import lax
from jax.experimental import pallas as pl
from jax.experimental.pallas import tpu as pltpu
from jax.experimental.pallas import tpu_sc as plsc

NUSER = 1000000
DIM = 64
B_TOTAL = 4096
L_TOTAL = 200
LANES = 16
NUM_CORES = 2
NUM_SUBCORES = 16
NW = NUM_CORES * NUM_SUBCORES
CHUNK = 512


def _sc_kernel(ids_hbm, tg_hbm, emb_hbm, tab_hbm, out_hbm,
               idx_v, tg_v, rows_v, emb_v, sem_g, sem_e):
    c = lax.axis_index("c")
    s = lax.axis_index("s")
    wid = s * NUM_CORES + c
    n_total = L_TOTAL * B_TOTAL
    n_per_w = n_total // NW
    nchunks = n_per_w // CHUNK
    n0 = wid * n_per_w

    iota = lax.iota(jnp.int32, LANES)

    def chunk_body(ch, carry):
        gn = n0 + ch * CHUNK
        l = gn // B_TOTAL
        b_hi0 = (gn - l * B_TOTAL) // 128

        pltpu.sync_copy(ids_hbm.at[pl.ds(gn, CHUNK)], idx_v)
        gather = pltpu.async_copy(tab_hbm.at[idx_v], rows_v, sem_g)
        emb_cp = pltpu.async_copy(
            emb_hbm.at[l, pl.ds(0, DIM // 8), pl.ds(b_hi0, CHUNK // 128)],
            emb_v, sem_e)
        tg_cp = pltpu.async_copy(tg_hbm.at[pl.ds(gn, CHUNK)], tg_v, sem_e)
        gather.wait()
        emb_cp.wait()
        tg_cp.wait()

        @plsc.parallel_loop(0, CHUNK // LANES)
        def pos_body(j):
            p = j * LANES
            t = tg_v[pl.ds(p, LANES)]
            row_idx = p + iota
            blk = j // 8
            off = (j - blk * 8) * LANES
            for grp in range(4):
                feats = range(grp * 16, grp * 16 + 16)
                shs = [plsc.load_gather(
                    rows_v, [row_idx, jnp.full((LANES,), d, jnp.int32)])
                       for d in feats]
                es = [emb_v[d // 8, blk, pl.ds((d % 8) * 128 + off, LANES)]
                      for d in feats]
                sigs = [1.0 / (1.0 + jnp.exp(-(t * sh))) for sh in shs]
                for i, d in enumerate(feats):
                    emb_v[d // 8, blk, pl.ds((d % 8) * 128 + off, LANES)] = (
                        es[i] * (1.0 + sigs[i]))
        pltpu.sync_copy(
            emb_v,
            out_hbm.at[l, pl.ds(0, DIM // 8), pl.ds(b_hi0, CHUNK // 128)])
        return carry

    lax.fori_loop(0, nchunks, chunk_body, 0)


@jax.jit
def _dspp_sc(ids_t, tg_t, emb_4d, table):
    mesh = plsc.VectorSubcoreMesh(core_axis_name="c", subcore_axis_name="s")
    run = pl.kernel(
        _sc_kernel,
        out_type=jax.ShapeDtypeStruct(
            (L_TOTAL, DIM // 8, B_TOTAL // 128, 1024), jnp.float32),
        mesh=mesh,
        scratch_types=[
            pltpu.VMEM((CHUNK,), jnp.int32),
            pltpu.VMEM((CHUNK,), jnp.float32),
            pltpu.VMEM((CHUNK, DIM), jnp.float32),
            pltpu.VMEM((DIM // 8, CHUNK // 128, 1024), jnp.float32),
            pltpu.SemaphoreType.DMA,
            pltpu.SemaphoreType.DMA,
        ],
        compiler_params=pltpu.CompilerParams(
            use_tc_tiling_on_sc=False, needs_layout_passes=False),
    )
    return run(ids_t, tg_t, emb_4d, table)


def kernel(id, emb, time_gap, user_shift_table):
    B, L = id.shape
    n = B * L
    emb_4d = (emb.transpose(1, 2, 0)
              .reshape(L, DIM // 8, 8, B // 128, 128)
              .transpose(0, 1, 3, 2, 4)
              .reshape(L, DIM // 8, B // 128, 1024))
    out_4d = _dspp_sc(
        id.T.reshape(n).astype(jnp.int32),
        time_gap.T.reshape(n),
        emb_4d,
        user_shift_table,
    )
    out = (out_4d.reshape(L, DIM // 8, B // 128, 8, 128)
           .transpose(0, 1, 3, 2, 4)
           .reshape(L, DIM, B)
           .transpose(2, 0, 1))
    return out


# speedup vs baseline: 5.2874x; 1.0718x over previous
import jax
import jax.numpy as jnp
from jax import lax
from jax.experimental import pallas as pl
from jax.experimental.pallas import tpu as pltpu
from jax.experimental.pallas import tpu_sc as plsc

NUSER = 1000000
DIM = 64
B_TOTAL = 4096
L_TOTAL = 200
LANES = 16
NUM_CORES = 2
NUM_SUBCORES = 16
NW = NUM_CORES * NUM_SUBCORES
CHUNK = 256


def _sc_kernel(ids_hbm, tg_hbm, emb_hbm, tab_hbm, out_hbm,
               idx0, idx1, tgv0, tgv1, rw0, rw1, em0, em1,
               sg0, sg1, sl0, sl1, so0, so1):
    idx = (idx0, idx1)
    tgv = (tgv0, tgv1)
    rw = (rw0, rw1)
    em = (em0, em1)
    sg = (sg0, sg1)
    sl = (sl0, sl1)
    so = (so0, so1)
    c = lax.axis_index("c")
    s = lax.axis_index("s")
    wid = s * NUM_CORES + c
    n_total = L_TOTAL * B_TOTAL
    n_per_w = n_total // NW
    nchunks = n_per_w // CHUNK
    n0 = wid * n_per_w

    iota = lax.iota(jnp.int32, LANES)

    def emb_slice(ci):
        gn = n0 + ci * CHUNK
        l = gn // B_TOTAL
        b_hi0 = (gn - l * B_TOTAL) // 128
        return l, b_hi0

    def issue(ci, b):
        gn = n0 + ci * CHUNK
        l, b_hi0 = emb_slice(ci)
        pltpu.sync_copy(ids_hbm.at[pl.ds(gn, CHUNK)], idx[b])
        pltpu.async_copy(tab_hbm.at[idx[b]], rw[b], sg[b])
        pltpu.async_copy(tg_hbm.at[pl.ds(gn, CHUNK)], tgv[b], sl[b])
        pltpu.async_copy(
            emb_hbm.at[l, pl.ds(0, DIM // 8), pl.ds(b_hi0, CHUNK // 128)],
            em[b], sl[b])

    def wait_in(ci, b):
        gn = n0 + ci * CHUNK
        l, b_hi0 = emb_slice(ci)
        pltpu.make_async_copy(tab_hbm.at[idx[b]], rw[b], sg[b]).wait()
        pltpu.make_async_copy(tg_hbm.at[pl.ds(gn, CHUNK)], tgv[b], sl[b]).wait()
        pltpu.make_async_copy(
            emb_hbm.at[l, pl.ds(0, DIM // 8), pl.ds(b_hi0, CHUNK // 128)],
            em[b], sl[b]).wait()

    def wait_out(ci, b):
        l, b_hi0 = emb_slice(ci)
        pltpu.make_async_copy(
            em[b],
            out_hbm.at[l, pl.ds(0, DIM // 8), pl.ds(b_hi0, CHUNK // 128)],
            so[b]).wait()

    def compute(b):
        @plsc.parallel_loop(0, CHUNK // LANES)
        def pos_body(j):
            p = j * LANES
            t = tgv[b][pl.ds(p, LANES)]
            row_idx = p + iota
            blk = j // 8
            off = (j - blk * 8) * LANES
            for grp in range(4):
                feats = range(grp * 16, grp * 16 + 16)
                shs = [plsc.load_gather(
                    rw[b], [row_idx, jnp.full((LANES,), d, jnp.int32)])
                       for d in feats]
                es = [em[b][d // 8, blk, pl.ds((d % 8) * 128 + off, LANES)]
                      for d in feats]
                sigs = [1.0 / (1.0 + jnp.exp(-(t * sh))) for sh in shs]
                for i, d in enumerate(feats):
                    em[b][d // 8, blk, pl.ds((d % 8) * 128 + off, LANES)] = (
                        es[i] * (1.0 + sigs[i]))

    issue(0, 0)

    def pair_body(i2, carry):
        for b in range(2):
            ci = i2 * 2 + b
            nb = 1 - b

            @pl.when(ci >= 1)
            def _():
                wait_out(ci - 1, nb)

            @pl.when(ci + 1 < nchunks)
            def _():
                issue(ci + 1, nb)

            wait_in(ci, b)
            compute(b)
            l, b_hi0 = emb_slice(ci)
            pltpu.async_copy(
                em[b],
                out_hbm.at[l, pl.ds(0, DIM // 8), pl.ds(b_hi0, CHUNK // 128)],
                so[b])
        return carry

    lax.fori_loop(0, nchunks // 2, pair_body, 0)
    wait_out(nchunks - 1, (nchunks - 1) % 2)


@jax.jit
def _dspp_sc(ids_t, tg_t, emb_4d, table):
    mesh = plsc.VectorSubcoreMesh(core_axis_name="c", subcore_axis_name="s")
    run = pl.kernel(
        _sc_kernel,
        out_type=jax.ShapeDtypeStruct(
            (L_TOTAL, DIM // 8, B_TOTAL // 128, 1024), jnp.float32),
        mesh=mesh,
        scratch_types=[
            pltpu.VMEM((CHUNK,), jnp.int32),
            pltpu.VMEM((CHUNK,), jnp.int32),
            pltpu.VMEM((CHUNK,), jnp.float32),
            pltpu.VMEM((CHUNK,), jnp.float32),
            pltpu.VMEM((CHUNK, DIM), jnp.float32),
            pltpu.VMEM((CHUNK, DIM), jnp.float32),
            pltpu.VMEM((DIM // 8, CHUNK // 128, 1024), jnp.float32),
            pltpu.VMEM((DIM // 8, CHUNK // 128, 1024), jnp.float32),
            pltpu.SemaphoreType.DMA,
            pltpu.SemaphoreType.DMA,
            pltpu.SemaphoreType.DMA,
            pltpu.SemaphoreType.DMA,
            pltpu.SemaphoreType.DMA,
            pltpu.SemaphoreType.DMA,
        ],
        compiler_params=pltpu.CompilerParams(
            use_tc_tiling_on_sc=False, needs_layout_passes=False),
    )
    return run(ids_t, tg_t, emb_4d, table)


def kernel(id, emb, time_gap, user_shift_table):
    B, L = id.shape
    n = B * L
    emb_4d = (emb.transpose(1, 2, 0)
              .reshape(L, DIM // 8, 8, B // 128, 128)
              .transpose(0, 1, 3, 2, 4)
              .reshape(L, DIM // 8, B // 128, 1024))
    out_4d = _dspp_sc(
        id.T.reshape(n).astype(jnp.int32),
        time_gap.T.reshape(n),
        emb_4d,
        user_shift_table,
    )
    out = (out_4d.reshape(L, DIM // 8, B // 128, 8, 128)
           .transpose(0, 1, 3, 2, 4)
           .reshape(L, DIM, B)
           .transpose(2, 0, 1))
    return out


# speedup vs baseline: 5.4928x; 1.0389x over previous
import jax
import jax.numpy as jnp
from jax import lax
from jax.experimental import pallas as pl
from jax.experimental.pallas import tpu as pltpu
from jax.experimental.pallas import tpu_sc as plsc

NUSER = 1000000
DIM = 64
B_TOTAL = 4096
L_TOTAL = 200
LANES = 16
NUM_CORES = 2
NUM_SUBCORES = 16
NW = NUM_CORES * NUM_SUBCORES
CHUNK = 256


def _sc_kernel(ids_hbm, tg_hbm, emb_hbm, tab_hbm, out_hbm,
               idx0, idx1, tgv0, tgv1, rw0, rw1, em0, em1,
               sg0, sg1, sl0, sl1, so0, so1, si0, si1):
    idx = (idx0, idx1)
    tgv = (tgv0, tgv1)
    rw = (rw0, rw1)
    em = (em0, em1)
    sg = (sg0, sg1)
    sl = (sl0, sl1)
    so = (so0, so1)
    si = (si0, si1)
    c = lax.axis_index("c")
    s = lax.axis_index("s")
    wid = s * NUM_CORES + c
    n_total = L_TOTAL * B_TOTAL
    n_per_w = n_total // NW
    nchunks = n_per_w // CHUNK
    n0 = wid * n_per_w

    iota = lax.iota(jnp.int32, LANES)

    def emb_slice(ci):
        gn = n0 + ci * CHUNK
        l = gn // B_TOTAL
        b_hi0 = (gn - l * B_TOTAL) // 128
        return l, b_hi0

    def issue_ids(ci, b):
        gn = n0 + ci * CHUNK
        pltpu.async_copy(ids_hbm.at[pl.ds(gn, CHUNK)], idx[b], si[b])

    def wait_ids(ci, b):
        gn = n0 + ci * CHUNK
        pltpu.make_async_copy(ids_hbm.at[pl.ds(gn, CHUNK)], idx[b],
                              si[b]).wait()

    def issue_rest(ci, b):
        gn = n0 + ci * CHUNK
        l, b_hi0 = emb_slice(ci)
        pltpu.async_copy(tab_hbm.at[idx[b]], rw[b], sg[b])
        pltpu.async_copy(tg_hbm.at[pl.ds(gn, CHUNK)], tgv[b], sl[b])
        pltpu.async_copy(
            emb_hbm.at[l, pl.ds(0, DIM // 8), pl.ds(b_hi0, CHUNK // 128)],
            em[b], sl[b])

    def wait_in(ci, b):
        gn = n0 + ci * CHUNK
        l, b_hi0 = emb_slice(ci)
        pltpu.make_async_copy(tab_hbm.at[idx[b]], rw[b], sg[b]).wait()
        pltpu.make_async_copy(tg_hbm.at[pl.ds(gn, CHUNK)], tgv[b], sl[b]).wait()
        pltpu.make_async_copy(
            emb_hbm.at[l, pl.ds(0, DIM // 8), pl.ds(b_hi0, CHUNK // 128)],
            em[b], sl[b]).wait()

    def wait_out(ci, b):
        l, b_hi0 = emb_slice(ci)
        pltpu.make_async_copy(
            em[b],
            out_hbm.at[l, pl.ds(0, DIM // 8), pl.ds(b_hi0, CHUNK // 128)],
            so[b]).wait()

    def compute(b):
        @plsc.parallel_loop(0, CHUNK // LANES)
        def pos_body(j):
            p = j * LANES
            t = tgv[b][pl.ds(p, LANES)]
            row_idx = p + iota
            blk = j // 8
            off = (j - blk * 8) * LANES
            for grp in range(4):
                feats = range(grp * 16, grp * 16 + 16)
                shs = [plsc.load_gather(
                    rw[b], [row_idx, jnp.full((LANES,), d, jnp.int32)])
                       for d in feats]
                es = [em[b][d // 8, blk, pl.ds((d % 8) * 128 + off, LANES)]
                      for d in feats]
                sigs = [1.0 / (1.0 + jnp.exp(-(t * sh))) for sh in shs]
                for i, d in enumerate(feats):
                    em[b][d // 8, blk, pl.ds((d % 8) * 128 + off, LANES)] = (
                        es[i] * (1.0 + sigs[i]))

    issue_ids(0, 0)
    wait_ids(0, 0)
    issue_rest(0, 0)
    issue_ids(1, 1)

    def pair_body(i2, carry):
        for b in range(2):
            ci = i2 * 2 + b
            nb = 1 - b

            @pl.when(ci >= 1)
            def _():
                wait_out(ci - 1, nb)

            @pl.when(ci + 1 < nchunks)
            def _():
                wait_ids(ci + 1, nb)
                issue_rest(ci + 1, nb)

            wait_in(ci, b)

            @pl.when(ci + 2 < nchunks)
            def _():
                issue_ids(ci + 2, b)

            compute(b)
            l, b_hi0 = emb_slice(ci)
            pltpu.async_copy(
                em[b],
                out_hbm.at[l, pl.ds(0, DIM // 8), pl.ds(b_hi0, CHUNK // 128)],
                so[b])
        return carry

    lax.fori_loop(0, nchunks // 2, pair_body, 0)
    wait_out(nchunks - 1, (nchunks - 1) % 2)


@jax.jit
def _dspp_sc(ids_t, tg_t, emb_4d, table):
    mesh = plsc.VectorSubcoreMesh(core_axis_name="c", subcore_axis_name="s")
    run = pl.kernel(
        _sc_kernel,
        out_type=jax.ShapeDtypeStruct(
            (L_TOTAL, DIM // 8, B_TOTAL // 128, 1024), jnp.float32),
        mesh=mesh,
        scratch_types=[
            pltpu.VMEM((CHUNK,), jnp.int32),
            pltpu.VMEM((CHUNK,), jnp.int32),
            pltpu.VMEM((CHUNK,), jnp.float32),
            pltpu.VMEM((CHUNK,), jnp.float32),
            pltpu.VMEM((CHUNK, DIM), jnp.float32),
            pltpu.VMEM((CHUNK, DIM), jnp.float32),
            pltpu.VMEM((DIM // 8, CHUNK // 128, 1024), jnp.float32),
            pltpu.VMEM((DIM // 8, CHUNK // 128, 1024), jnp.float32),
            pltpu.SemaphoreType.DMA,
            pltpu.SemaphoreType.DMA,
            pltpu.SemaphoreType.DMA,
            pltpu.SemaphoreType.DMA,
            pltpu.SemaphoreType.DMA,
            pltpu.SemaphoreType.DMA,
            pltpu.SemaphoreType.DMA,
            pltpu.SemaphoreType.DMA,
        ],
        compiler_params=pltpu.CompilerParams(
            use_tc_tiling_on_sc=False, needs_layout_passes=False),
    )
    return run(ids_t, tg_t, emb_4d, table)


def kernel(id, emb, time_gap, user_shift_table):
    B, L = id.shape
    n = B * L
    emb_4d = (emb.transpose(1, 2, 0)
              .reshape(L, DIM // 8, 8, B // 128, 128)
              .transpose(0, 1, 3, 2, 4)
              .reshape(L, DIM // 8, B // 128, 1024))
    out_4d = _dspp_sc(
        id.T.reshape(n).astype(jnp.int32),
        time_gap.T.reshape(n),
        emb_4d,
        user_shift_table,
    )
    out = (out_4d.reshape(L, DIM // 8, B // 128, 8, 128)
           .transpose(0, 1, 3, 2, 4)
           .reshape(L, DIM, B)
           .transpose(2, 0, 1))
    return out
